# baseline (speedup 1.0000x reference)
import functools

import jax
import jax.numpy as jnp
from jax.experimental import pallas as pl
from jax.experimental.pallas import tpu as pltpu

LEAKY_SLOPE = 0.2
BN_EPS = 1e-5


def _wmat(w, C):
    Co, Ci = w.shape[0], w.shape[1]
    Wt = jnp.transpose(w, (2, 3, 1, 0)).astype(jnp.float32)
    Wt = jnp.pad(Wt, ((0, 0), (0, 0), (0, C - Ci), (0, 0)))
    Wt = Wt.reshape(2, 2, 2, 2, C, Co).transpose(0, 2, 1, 3, 4, 5)
    return Wt.reshape(4, 4 * C, Co)


def _bn_coeffs(stats, m_rows, g, be):
    st = stats.reshape(-1, 8, stats.shape[-1])
    ssum = jnp.sum(st[:, 0, :], axis=0)
    ssq = jnp.sum(st[:, 1, :], axis=0)
    mean = ssum / m_rows
    var = jnp.maximum(ssq / m_rows - mean * mean, 0.0)
    s = g * jax.lax.rsqrt(var + BN_EPS)
    t = be - mean * s
    return s, t


def _taps_matmul(zb, b_ref, Ho):
    NB, _, _, K4 = zb.shape
    acc = None
    for t, (di, dj) in enumerate(((0, 0), (0, 1), (1, 0), (1, 1))):
        a = zb[:, di:di + Ho, dj:dj + Ho, :].reshape(NB * Ho * Ho, K4)
        d = jnp.dot(a, b_ref[t], preferred_element_type=jnp.float32)
        acc = d if acc is None else acc + d
    return acc


def _emit_z(y4):
    nb, H, _, C = y4.shape
    Z = H // 2 + 1
    yr = y4.reshape(nb, Z - 1, 2, H, C)
    zrow = jnp.zeros((nb, 1, H, C), jnp.float32)
    rows0 = jnp.concatenate([zrow, yr[:, :, 1]], axis=1)
    rows1 = jnp.concatenate([yr[:, :, 0], zrow], axis=1)
    planes = []
    for rows in (rows0, rows1):
        cc = rows.reshape(nb, Z, Z - 1, 2, C)
        zcol = jnp.zeros((nb, Z, 1, C), jnp.float32)
        planes.append(jnp.concatenate([zcol, cc[:, :, :, 1]], axis=2))
        planes.append(jnp.concatenate([cc[:, :, :, 0], zcol], axis=2))
    return jnp.concatenate(planes, axis=-1)


def _border_corr(w, s_prev, t_prev, Ho):
    del s_prev
    vb = jnp.where(t_prev > 0, t_prev, LEAKY_SLOPE * t_prev)
    ws = w.astype(jnp.float32)
    r = [jnp.einsum('c,ock->o', vb, ws[:, :, kh, :]) for kh in (0, 3)]
    cl = [jnp.einsum('c,ock->o', vb, ws[:, :, :, kw]) for kw in (0, 3)]
    d = [[jnp.einsum('c,oc->o', vb, ws[:, :, kh, kw]) for kw in (0, 3)]
         for kh in (0, 3)]
    Co = ws.shape[0]
    E = jnp.zeros((Ho, Ho, Co), jnp.float32)
    E = E.at[0, :, :].add(r[0]).at[Ho - 1, :, :].add(r[1])
    E = E.at[:, 0, :].add(cl[0]).at[:, Ho - 1, :].add(cl[1])
    E = E.at[0, 0].add(-d[0][0]).at[0, Ho - 1].add(-d[0][1])
    E = E.at[Ho - 1, 0].add(-d[1][0]).at[Ho - 1, Ho - 1].add(-d[1][1])
    return E.reshape(Ho * Ho, Co)


def _l1_kernel(xq_ref, t_ref, bias_ref, zo_ref, *, nb):
    xb = xq_ref[...]
    acc = None
    for di in (0, 1):
        for qi in (0, 1):
            a = jnp.concatenate(
                [xb[:, qi, c, di:di + 16, :] for c in range(3)],
                axis=-1).reshape(nb * 16, 102)
            d = jnp.dot(a, t_ref[2 * di + qi],
                        preferred_element_type=jnp.float32)
            acc = d if acc is None else acc + d
    y = acc + bias_ref[...]
    y = jnp.where(y > 0, y, LEAKY_SLOPE * y)
    zo_ref[...] = _emit_z(y.reshape(nb, 16, 16, 64))


def _conv_kernel(z_ref, b_ref, zo_ref, st_ref, *, Ho, nb):
    acc = _taps_matmul(z_ref[...], b_ref, Ho)
    st_ref[0:1, :] = jnp.sum(acc, axis=0, keepdims=True)
    st_ref[1:2, :] = jnp.sum(acc * acc, axis=0, keepdims=True)
    zo_ref[...] = _emit_z(acc.reshape(nb, Ho, Ho, acc.shape[-1]))


def _affine_conv_kernel(z_ref, b_ref, s_ref, t_ref, e_ref, zo_ref, st_ref, *,
                        Ho, nb):
    zb = z_ref[...]
    y = zb * s_ref[...] + t_ref[...]
    y = jnp.where(y > 0, y, LEAKY_SLOPE * y)
    acc = _taps_matmul(y, b_ref, Ho)
    Co = acc.shape[-1]
    acc = (acc.reshape(nb, Ho * Ho, Co) - e_ref[...]).reshape(nb * Ho * Ho, Co)
    st_ref[0:1, :] = jnp.sum(acc, axis=0, keepdims=True)
    st_ref[1:2, :] = jnp.sum(acc * acc, axis=0, keepdims=True)
    zo_ref[...] = _emit_z(acc.reshape(nb, Ho, Ho, Co))


def _l4_kernel(z_ref, b_ref, s_ref, t_ref, e_ref, zo_ref, st_ref, *, nb):
    zb = z_ref[...]
    y = zb * s_ref[...] + t_ref[...]
    y = jnp.where(y > 0, y, LEAKY_SLOPE * y)
    acc = _taps_matmul(y, b_ref, 2)
    acc = (acc.reshape(nb, 4, 512) - e_ref[...]).reshape(nb * 4, 512)
    st_ref[0:1, :] = jnp.sum(acc, axis=0, keepdims=True)
    st_ref[1:2, :] = jnp.sum(acc * acc, axis=0, keepdims=True)
    z5 = _emit_z(acc.reshape(nb, 2, 2, 512))
    zo_ref[...] = z5.reshape(nb, 8192)


def _l5_kernel(a_ref, b_ref, s_ref, t_ref, bias_ref, o_ref, acc_ref):
    k = pl.program_id(1)

    @pl.when(k == 0)
    def _():
        acc_ref[...] = jnp.zeros_like(acc_ref)

    z = a_ref[...] * s_ref[...] + t_ref[...]
    z = jnp.where(z > 0, z, LEAKY_SLOPE * z)
    acc_ref[...] += jnp.dot(z, b_ref[...], preferred_element_type=jnp.float32)

    @pl.when(k == pl.num_programs(1) - 1)
    def _():
        y = acc_ref[...] + bias_ref[...]
        o_ref[...] = 1.0 / (1.0 + jnp.exp(-y))


def kernel(x, w1, b1, w2, g2, be2, w3, g3, be3, w4, g4, be4, w5, b5):
    N = x.shape[0]

    xpad = jnp.pad(x.astype(jnp.float32), ((0, 0), (0, 0), (1, 1), (1, 1)))
    xq = xpad.reshape(N, 3, 17, 2, 34).transpose(0, 3, 1, 2, 4)
    w1f = w1.astype(jnp.float32)
    tsel = jnp.zeros((4, 3, 34, 16, 64), jnp.float32)
    ow = jnp.arange(16)
    for kw in range(4):
        upd = jnp.broadcast_to(
            jnp.transpose(w1f[:, :, :, kw], (2, 1, 0))[:, :, None, :],
            (4, 3, 16, 64))
        tsel = tsel.at[:, :, 2 * ow + kw, ow, :].add(upd)
    tsel = tsel.reshape(4, 102, 1024)
    bias1 = jnp.tile(b1.astype(jnp.float32), 16).reshape(1, 1024)

    nb1 = min(64, N)
    z2 = pl.pallas_call(
        functools.partial(_l1_kernel, nb=nb1),
        out_shape=jax.ShapeDtypeStruct((N, 9, 9, 256), jnp.float32),
        grid=(N // nb1,),
        in_specs=[
            pl.BlockSpec((nb1, 2, 3, 17, 34), lambda m: (m, 0, 0, 0, 0)),
            pl.BlockSpec((4, 102, 1024), lambda m: (0, 0, 0)),
            pl.BlockSpec((1, 1024), lambda m: (0, 0)),
        ],
        out_specs=pl.BlockSpec((nb1, 9, 9, 256), lambda m: (m, 0, 0, 0)),
        compiler_params=pltpu.CompilerParams(
            dimension_semantics=("parallel",)),
    )(xq, tsel, bias1)

    nb2 = min(64, N)
    z3, st2 = pl.pallas_call(
        functools.partial(_conv_kernel, Ho=8, nb=nb2),
        out_shape=(jax.ShapeDtypeStruct((N, 5, 5, 512), jnp.float32),
                   jax.ShapeDtypeStruct((N // nb2 * 8, 128), jnp.float32)),
        grid=(N // nb2,),
        in_specs=[
            pl.BlockSpec((nb2, 9, 9, 256), lambda m: (m, 0, 0, 0)),
            pl.BlockSpec((4, 256, 128), lambda m: (0, 0, 0)),
        ],
        out_specs=(pl.BlockSpec((nb2, 5, 5, 512), lambda m: (m, 0, 0, 0)),
                   pl.BlockSpec((8, 128), lambda m: (m, 0))),
        compiler_params=pltpu.CompilerParams(
            dimension_semantics=("parallel",)),
    )(z2, _wmat(w2, 64))
    s2, t2 = _bn_coeffs(st2, N * 64, g2, be2)

    nb3 = min(128, N)
    z4, st3 = pl.pallas_call(
        functools.partial(_affine_conv_kernel, Ho=4, nb=nb3),
        out_shape=(jax.ShapeDtypeStruct((N, 3, 3, 1024), jnp.float32),
                   jax.ShapeDtypeStruct((N // nb3 * 8, 256), jnp.float32)),
        grid=(N // nb3,),
        in_specs=[
            pl.BlockSpec((nb3, 5, 5, 512), lambda m: (m, 0, 0, 0)),
            pl.BlockSpec((4, 512, 256), lambda m: (0, 0, 0)),
            pl.BlockSpec((1, 512), lambda m: (0, 0)),
            pl.BlockSpec((1, 512), lambda m: (0, 0)),
            pl.BlockSpec((16, 256), lambda m: (0, 0)),
        ],
        out_specs=(pl.BlockSpec((nb3, 3, 3, 1024), lambda m: (m, 0, 0, 0)),
                   pl.BlockSpec((8, 256), lambda m: (m, 0))),
        compiler_params=pltpu.CompilerParams(
            dimension_semantics=("parallel",)),
    )(z3, _wmat(w3, 128),
      jnp.tile(s2, 4).reshape(1, 512), jnp.tile(t2, 4).reshape(1, 512),
      _border_corr(w3, s2, t2, 4))
    s3, t3 = _bn_coeffs(st3, N * 16, g3, be3)

    nb4 = min(128, N)
    z5, st4 = pl.pallas_call(
        functools.partial(_l4_kernel, nb=nb4),
        out_shape=(jax.ShapeDtypeStruct((N, 8192), jnp.float32),
                   jax.ShapeDtypeStruct((N // nb4 * 8, 512), jnp.float32)),
        grid=(N // nb4,),
        in_specs=[
            pl.BlockSpec((nb4, 3, 3, 1024), lambda m: (m, 0, 0, 0)),
            pl.BlockSpec((4, 1024, 512), lambda m: (0, 0, 0)),
            pl.BlockSpec((1, 1024), lambda m: (0, 0)),
            pl.BlockSpec((1, 1024), lambda m: (0, 0)),
            pl.BlockSpec((4, 512), lambda m: (0, 0)),
        ],
        out_specs=(pl.BlockSpec((nb4, 8192), lambda m: (m, 0)),
                   pl.BlockSpec((8, 512), lambda m: (m, 0))),
        compiler_params=pltpu.CompilerParams(
            dimension_semantics=("parallel",)),
    )(z4, _wmat(w4, 256),
      jnp.tile(s3, 4).reshape(1, 1024), jnp.tile(t3, 4).reshape(1, 1024),
      _border_corr(w4, s3, t3, 2))
    s4, t4 = _bn_coeffs(st4, N * 4, g4, be4)

    b5m = jnp.pad(_wmat(w5, 512).reshape(8192, 1), ((0, 0), (0, 127)))
    bias5 = jnp.pad(b5.astype(jnp.float32), (0, 127)).reshape(1, 128)
    ll = jnp.arange(8192)
    zi5, zj5 = ll // 4096, (ll // 2048) % 2
    qi5, qj5 = (ll // 1024) % 2, (ll // 512) % 2
    live = jnp.logical_not(((zi5 == 0) & (qi5 == 0)) | ((zi5 == 1) & (qi5 == 1))
                           | ((zj5 == 0) & (qj5 == 0)) | ((zj5 == 1) & (qj5 == 1))
                           ).astype(jnp.float32)
    s4z = (jnp.tile(s4, 16) * live).reshape(1, 8192)
    t4z = (jnp.tile(t4, 16) * live).reshape(1, 8192)

    nb5 = N // 2
    y = pl.pallas_call(
        _l5_kernel,
        out_shape=jax.ShapeDtypeStruct((N, 128), jnp.float32),
        grid=(2, 4),
        in_specs=[
            pl.BlockSpec((nb5, 2048), lambda m, k: (m, k)),
            pl.BlockSpec((2048, 128), lambda m, k: (k, 0)),
            pl.BlockSpec((1, 2048), lambda m, k: (0, k)),
            pl.BlockSpec((1, 2048), lambda m, k: (0, k)),
            pl.BlockSpec((1, 128), lambda m, k: (0, 0)),
        ],
        out_specs=pl.BlockSpec((nb5, 128), lambda m, k: (m, 0)),
        scratch_shapes=[pltpu.VMEM((nb5, 128), jnp.float32)],
        compiler_params=pltpu.CompilerParams(
            dimension_semantics=("parallel", "arbitrary")),
    )(z5, b5m, s4z, t4z, bias5)

    return y[:, :1].reshape(N, 1, 1, 1)


# speedup vs baseline: 1.1851x; 1.1851x over previous
import functools

import jax
import jax.numpy as jnp
from jax.experimental import pallas as pl
from jax.experimental.pallas import tpu as pltpu

LEAKY_SLOPE = 0.2
BN_EPS = 1e-5


def _wmat(w, C):
    Co, Ci = w.shape[0], w.shape[1]
    Wt = jnp.transpose(w, (2, 3, 1, 0)).astype(jnp.float32)
    Wt = jnp.pad(Wt, ((0, 0), (0, 0), (0, C - Ci), (0, 0)))
    Wt = Wt.reshape(2, 2, 2, 2, C, Co).transpose(0, 2, 1, 3, 4, 5)
    return Wt.reshape(4, 4 * C, Co)


def _bn_coeffs(st, g, be, m_rows):
    stp = st.reshape(-1, 8, st.shape[-1])
    ssum = jnp.sum(stp[:, 0:1, :], axis=0)
    ssq = jnp.sum(stp[:, 1:2, :], axis=0)
    mean = ssum / m_rows
    var = jnp.maximum(ssq / m_rows - mean * mean, 0.0)
    s = g * jax.lax.rsqrt(var + BN_EPS)
    t = be - mean * s
    return s, t


def _taps_matmul(zb, b_ref, Ho):
    NB, _, _, K4 = zb.shape
    acc = None
    for t, (di, dj) in enumerate(((0, 0), (0, 1), (1, 0), (1, 1))):
        a = zb[:, di:di + Ho, dj:dj + Ho, :].reshape(NB * Ho * Ho, K4)
        d = jnp.dot(a, b_ref[t], preferred_element_type=jnp.float32)
        acc = d if acc is None else acc + d
    return acc


def _emit_z(y4):
    nb, H, _, C = y4.shape
    Z = H // 2 + 1
    yr = y4.reshape(nb, Z - 1, 2, H, C)
    zrow = jnp.zeros((nb, 1, H, C), jnp.float32)
    rows0 = jnp.concatenate([zrow, yr[:, :, 1]], axis=1)
    rows1 = jnp.concatenate([yr[:, :, 0], zrow], axis=1)
    planes = []
    for rows in (rows0, rows1):
        cc = rows.reshape(nb, Z, Z - 1, 2, C)
        zcol = jnp.zeros((nb, Z, 1, C), jnp.float32)
        planes.append(jnp.concatenate([zcol, cc[:, :, :, 1]], axis=2))
        planes.append(jnp.concatenate([cc[:, :, :, 0], zcol], axis=2))
    return jnp.concatenate(planes, axis=-1)


def _border_mask(Z, C4, C):
    zi = jax.lax.broadcasted_iota(jnp.int32, (Z, Z, C4), 0)
    zj = jax.lax.broadcasted_iota(jnp.int32, (Z, Z, C4), 1)
    ll = jax.lax.broadcasted_iota(jnp.int32, (Z, Z, C4), 2)
    qi = ll // (2 * C)
    qj = (ll // C) % 2
    border = ((zi == 0) & (qi == 0)) | ((zi == Z - 1) & (qi == 1)) \
        | ((zj == 0) & (qj == 0)) | ((zj == Z - 1) & (qj == 1))
    return jnp.where(border, 0.0, 1.0).astype(jnp.float32)


def _l1_kernel(xq_ref, t_ref, bias_ref, zo_ref, *, nb):
    xb = xq_ref[...]
    acc = None
    for di in (0, 1):
        for qi in (0, 1):
            a = jnp.concatenate(
                [xb[:, qi, c, di:di + 16, :] for c in range(3)],
                axis=-1).reshape(nb * 16, 102)
            d = jnp.dot(a, t_ref[2 * di + qi],
                        preferred_element_type=jnp.float32)
            acc = d if acc is None else acc + d
    y = acc + bias_ref[...]
    y = jnp.where(y > 0, y, LEAKY_SLOPE * y)
    zo_ref[...] = _emit_z(y.reshape(nb, 16, 16, 64))


def _conv_kernel(z_ref, b_ref, zo_ref, st_ref, *, Ho, nb):
    acc = _taps_matmul(z_ref[...], b_ref, Ho)
    st_ref[0:1, :] = jnp.sum(acc, axis=0, keepdims=True)
    st_ref[1:2, :] = jnp.sum(acc * acc, axis=0, keepdims=True)
    zo_ref[...] = _emit_z(acc.reshape(nb, Ho, Ho, acc.shape[-1]))


def _affine_conv_kernel(z_ref, b_ref, stin_ref, g_ref, be_ref, zo_ref,
                        st_ref, *, Ho, nb, C, m_rows):
    zb = z_ref[...]
    s, t = _bn_coeffs(stin_ref[...], g_ref[...], be_ref[...], m_rows)
    s4 = jnp.concatenate([s] * 4, axis=-1)
    t4 = jnp.concatenate([t] * 4, axis=-1)
    y = zb * s4 + t4
    y = jnp.where(y > 0, y, LEAKY_SLOPE * y)
    y = y * _border_mask(zb.shape[1], zb.shape[-1], C)
    acc = _taps_matmul(y, b_ref, Ho)
    st_ref[0:1, :] = jnp.sum(acc, axis=0, keepdims=True)
    st_ref[1:2, :] = jnp.sum(acc * acc, axis=0, keepdims=True)
    out = _emit_z(acc.reshape(nb, Ho, Ho, acc.shape[-1]))
    zo_ref[...] = out.reshape(zo_ref.shape)


def _l5_kernel(a_ref, b_ref, stin_ref, g_ref, be_ref, bias_ref, o_ref,
               acc_ref, *, m_rows):
    k = pl.program_id(1)

    @pl.when(k == 0)
    def _():
        acc_ref[...] = jnp.zeros_like(acc_ref)

    s, t = _bn_coeffs(stin_ref[...], g_ref[...], be_ref[...], m_rows)
    s4 = jnp.concatenate([s] * 4, axis=-1)
    t4 = jnp.concatenate([t] * 4, axis=-1)
    ll = jax.lax.broadcasted_iota(jnp.int32, (1, 2048), 1)
    qi, qj = (ll // 1024) % 2, (ll // 512) % 2
    border = ((k // 2 == 0) & (qi == 0)) | ((k // 2 == 1) & (qi == 1)) \
        | ((k % 2 == 0) & (qj == 0)) | ((k % 2 == 1) & (qj == 1))
    z = a_ref[...] * s4 + t4
    z = jnp.where(z > 0, z, LEAKY_SLOPE * z)
    z = jnp.where(border, 0.0, z)
    acc_ref[...] += jnp.dot(z, b_ref[...], preferred_element_type=jnp.float32)

    @pl.when(k == pl.num_programs(1) - 1)
    def _():
        y = acc_ref[...] + bias_ref[...]
        o_ref[...] = 1.0 / (1.0 + jnp.exp(-y))


def kernel(x, w1, b1, w2, g2, be2, w3, g3, be3, w4, g4, be4, w5, b5):
    N = x.shape[0]

    xpad = jnp.pad(x.astype(jnp.float32), ((0, 0), (0, 0), (1, 1), (1, 1)))
    xq = xpad.reshape(N, 3, 17, 2, 34).transpose(0, 3, 1, 2, 4)
    w1f = w1.astype(jnp.float32)
    tsel = jnp.zeros((4, 3, 34, 16, 64), jnp.float32)
    ow = jnp.arange(16)
    for kw in range(4):
        upd = jnp.broadcast_to(
            jnp.transpose(w1f[:, :, :, kw], (2, 1, 0))[:, :, None, :],
            (4, 3, 16, 64))
        tsel = tsel.at[:, :, 2 * ow + kw, ow, :].add(upd)
    tsel = tsel.reshape(4, 102, 1024)
    bias1 = jnp.tile(b1.astype(jnp.float32), 16).reshape(1, 1024)

    nb1 = min(64, N)
    z2 = pl.pallas_call(
        functools.partial(_l1_kernel, nb=nb1),
        out_shape=jax.ShapeDtypeStruct((N, 9, 9, 256), jnp.float32),
        grid=(N // nb1,),
        in_specs=[
            pl.BlockSpec((nb1, 2, 3, 17, 34), lambda m: (m, 0, 0, 0, 0)),
            pl.BlockSpec((4, 102, 1024), lambda m: (0, 0, 0)),
            pl.BlockSpec((1, 1024), lambda m: (0, 0)),
        ],
        out_specs=pl.BlockSpec((nb1, 9, 9, 256), lambda m: (m, 0, 0, 0)),
        compiler_params=pltpu.CompilerParams(
            dimension_semantics=("parallel",)),
    )(xq, tsel, bias1)

    nb2 = min(64, N)
    z3, st2 = pl.pallas_call(
        functools.partial(_conv_kernel, Ho=8, nb=nb2),
        out_shape=(jax.ShapeDtypeStruct((N, 5, 5, 512), jnp.float32),
                   jax.ShapeDtypeStruct((N // nb2 * 8, 128), jnp.float32)),
        grid=(N // nb2,),
        in_specs=[
            pl.BlockSpec((nb2, 9, 9, 256), lambda m: (m, 0, 0, 0)),
            pl.BlockSpec((4, 256, 128), lambda m: (0, 0, 0)),
        ],
        out_specs=(pl.BlockSpec((nb2, 5, 5, 512), lambda m: (m, 0, 0, 0)),
                   pl.BlockSpec((8, 128), lambda m: (m, 0))),
        compiler_params=pltpu.CompilerParams(
            dimension_semantics=("parallel",)),
    )(z2, _wmat(w2, 64))
    nblk2 = N // nb2

    nb3 = min(128, N)
    z4, st3 = pl.pallas_call(
        functools.partial(_affine_conv_kernel, Ho=4, nb=nb3, C=128,
                          m_rows=float(N * 64)),
        out_shape=(jax.ShapeDtypeStruct((N, 3, 3, 1024), jnp.float32),
                   jax.ShapeDtypeStruct((N // nb3 * 8, 256), jnp.float32)),
        grid=(N // nb3,),
        in_specs=[
            pl.BlockSpec((nb3, 5, 5, 512), lambda m: (m, 0, 0, 0)),
            pl.BlockSpec((4, 512, 256), lambda m: (0, 0, 0)),
            pl.BlockSpec((nblk2 * 8, 128), lambda m: (0, 0)),
            pl.BlockSpec((1, 128), lambda m: (0, 0)),
            pl.BlockSpec((1, 128), lambda m: (0, 0)),
        ],
        out_specs=(pl.BlockSpec((nb3, 3, 3, 1024), lambda m: (m, 0, 0, 0)),
                   pl.BlockSpec((8, 256), lambda m: (m, 0))),
        compiler_params=pltpu.CompilerParams(
            dimension_semantics=("parallel",)),
    )(z3, _wmat(w3, 128), st2,
      g2.astype(jnp.float32).reshape(1, 128),
      be2.astype(jnp.float32).reshape(1, 128))
    nblk3 = N // nb3

    nb4 = min(128, N)
    z5, st4 = pl.pallas_call(
        functools.partial(_affine_conv_kernel, Ho=2, nb=nb4, C=256,
                          m_rows=float(N * 16)),
        out_shape=(jax.ShapeDtypeStruct((N, 8192), jnp.float32),
                   jax.ShapeDtypeStruct((N // nb4 * 8, 512), jnp.float32)),
        grid=(N // nb4,),
        in_specs=[
            pl.BlockSpec((nb4, 3, 3, 1024), lambda m: (m, 0, 0, 0)),
            pl.BlockSpec((4, 1024, 512), lambda m: (0, 0, 0)),
            pl.BlockSpec((nblk3 * 8, 256), lambda m: (0, 0)),
            pl.BlockSpec((1, 256), lambda m: (0, 0)),
            pl.BlockSpec((1, 256), lambda m: (0, 0)),
        ],
        out_specs=(pl.BlockSpec((nb4, 8192), lambda m: (m, 0)),
                   pl.BlockSpec((8, 512), lambda m: (m, 0))),
        compiler_params=pltpu.CompilerParams(
            dimension_semantics=("parallel",)),
    )(z4, _wmat(w4, 256), st3,
      g3.astype(jnp.float32).reshape(1, 256),
      be3.astype(jnp.float32).reshape(1, 256))
    nblk4 = N // nb4

    b5m = jnp.pad(_wmat(w5, 512).reshape(8192, 1), ((0, 0), (0, 127)))
    bias5 = jnp.pad(b5.astype(jnp.float32), (0, 127)).reshape(1, 128)

    nb5 = N // 2
    y = pl.pallas_call(
        functools.partial(_l5_kernel, m_rows=float(N * 4)),
        out_shape=jax.ShapeDtypeStruct((N, 128), jnp.float32),
        grid=(2, 4),
        in_specs=[
            pl.BlockSpec((nb5, 2048), lambda m, k: (m, k)),
            pl.BlockSpec((2048, 128), lambda m, k: (k, 0)),
            pl.BlockSpec((nblk4 * 8, 512), lambda m, k: (0, 0)),
            pl.BlockSpec((1, 512), lambda m, k: (0, 0)),
            pl.BlockSpec((1, 512), lambda m, k: (0, 0)),
            pl.BlockSpec((1, 128), lambda m, k: (0, 0)),
        ],
        out_specs=pl.BlockSpec((nb5, 128), lambda m, k: (m, 0)),
        scratch_shapes=[pltpu.VMEM((nb5, 128), jnp.float32)],
        compiler_params=pltpu.CompilerParams(
            dimension_semantics=("parallel", "arbitrary")),
    )(z5, b5m, st4,
      g4.astype(jnp.float32).reshape(1, 512),
      be4.astype(jnp.float32).reshape(1, 512), bias5)

    return y[:, :1].reshape(N, 1, 1, 1)


# speedup vs baseline: 1.1853x; 1.0002x over previous
import functools

import jax
import jax.numpy as jnp
from jax.experimental import pallas as pl
from jax.experimental.pallas import tpu as pltpu

LEAKY_SLOPE = 0.2
BN_EPS = 1e-5


def _wmat(w, C):
    Co, Ci = w.shape[0], w.shape[1]
    Wt = jnp.transpose(w, (2, 3, 1, 0)).astype(jnp.float32)
    Wt = jnp.pad(Wt, ((0, 0), (0, 0), (0, C - Ci), (0, 0)))
    Wt = Wt.reshape(2, 2, 2, 2, C, Co).transpose(0, 2, 1, 3, 4, 5)
    return Wt.reshape(4, 4 * C, Co)


def _bn_coeffs(st, g, be, m_rows):
    stp = st.reshape(-1, 8, st.shape[-1])
    ssum = jnp.sum(stp[:, 0:1, :], axis=0)
    ssq = jnp.sum(stp[:, 1:2, :], axis=0)
    mean = ssum / m_rows
    var = jnp.maximum(ssq / m_rows - mean * mean, 0.0)
    s = g * jax.lax.rsqrt(var + BN_EPS)
    t = be - mean * s
    return s, t


def _taps_matmul(zb, b_ref, Ho):
    NB, _, _, K4 = zb.shape
    acc = None
    for t, (di, dj) in enumerate(((0, 0), (0, 1), (1, 0), (1, 1))):
        a = zb[:, di:di + Ho, dj:dj + Ho, :].reshape(NB * Ho * Ho, K4)
        d = jnp.dot(a, b_ref[t], preferred_element_type=jnp.float32)
        acc = d if acc is None else acc + d
    return acc


def _emit_z(y4):
    nb, H, _, C = y4.shape
    Z = H // 2 + 1
    yr = y4.reshape(nb, Z - 1, 2, H, C)
    zrow = jnp.zeros((nb, 1, H, C), jnp.float32)
    rows0 = jnp.concatenate([zrow, yr[:, :, 1]], axis=1)
    rows1 = jnp.concatenate([yr[:, :, 0], zrow], axis=1)
    planes = []
    for rows in (rows0, rows1):
        cc = rows.reshape(nb, Z, Z - 1, 2, C)
        zcol = jnp.zeros((nb, Z, 1, C), jnp.float32)
        planes.append(jnp.concatenate([zcol, cc[:, :, :, 1]], axis=2))
        planes.append(jnp.concatenate([cc[:, :, :, 0], zcol], axis=2))
    return jnp.concatenate(planes, axis=-1)


def _border_mask(Z, C4, C):
    zi = jax.lax.broadcasted_iota(jnp.int32, (Z, Z, C4), 0)
    zj = jax.lax.broadcasted_iota(jnp.int32, (Z, Z, C4), 1)
    ll = jax.lax.broadcasted_iota(jnp.int32, (Z, Z, C4), 2)
    qi = ll // (2 * C)
    qj = (ll // C) % 2
    border = ((zi == 0) & (qi == 0)) | ((zi == Z - 1) & (qi == 1)) \
        | ((zj == 0) & (qj == 0)) | ((zj == Z - 1) & (qj == 1))
    return jnp.where(border, 0.0, 1.0).astype(jnp.float32)


def _l1_kernel(xq_ref, t_ref, bias_ref, zo_ref, *, nb):
    xb = xq_ref[...]
    acc = None
    for di in (0, 1):
        for qi in (0, 1):
            a = jnp.concatenate(
                [xb[:, qi, c, di:di + 16, :] for c in range(3)],
                axis=-1).reshape(nb * 16, 102)
            d = jnp.dot(a, t_ref[2 * di + qi],
                        preferred_element_type=jnp.float32)
            acc = d if acc is None else acc + d
    y = acc + bias_ref[...]
    y = jnp.where(y > 0, y, LEAKY_SLOPE * y)
    zo_ref[...] = _emit_z(y.reshape(nb, 16, 16, 64))


def _conv_kernel(z_ref, b_ref, zo_ref, st_ref, *, Ho, nb):
    acc = _taps_matmul(z_ref[...], b_ref, Ho)
    st_ref[0:1, :] = jnp.sum(acc, axis=0, keepdims=True)
    st_ref[1:2, :] = jnp.sum(acc * acc, axis=0, keepdims=True)
    zo_ref[...] = _emit_z(acc.reshape(nb, Ho, Ho, acc.shape[-1]))


def _affine_conv_kernel(z_ref, b_ref, stin_ref, g_ref, be_ref, zo_ref,
                        st_ref, *, Ho, nb, C, m_rows):
    zb = z_ref[...]
    s, t = _bn_coeffs(stin_ref[...], g_ref[...], be_ref[...], m_rows)
    s4 = jnp.concatenate([s] * 4, axis=-1)
    t4 = jnp.concatenate([t] * 4, axis=-1)
    y = zb * s4 + t4
    y = jnp.where(y > 0, y, LEAKY_SLOPE * y)
    y = y * _border_mask(zb.shape[1], zb.shape[-1], C)
    acc = _taps_matmul(y, b_ref, Ho)
    st_ref[0:1, :] = jnp.sum(acc, axis=0, keepdims=True)
    st_ref[1:2, :] = jnp.sum(acc * acc, axis=0, keepdims=True)
    out = _emit_z(acc.reshape(nb, Ho, Ho, acc.shape[-1]))
    zo_ref[...] = out.reshape(zo_ref.shape)


def _l5_kernel(a_ref, b_ref, stin_ref, g_ref, be_ref, bias_ref, o_ref,
               acc_ref, *, m_rows):
    k = pl.program_id(1)

    @pl.when(k == 0)
    def _():
        acc_ref[...] = jnp.zeros_like(acc_ref)

    s, t = _bn_coeffs(stin_ref[...], g_ref[...], be_ref[...], m_rows)
    s4 = jnp.concatenate([s] * 4, axis=-1)
    t4 = jnp.concatenate([t] * 4, axis=-1)
    ll = jax.lax.broadcasted_iota(jnp.int32, (1, 2048), 1)
    qi, qj = (ll // 1024) % 2, (ll // 512) % 2
    border = ((k // 2 == 0) & (qi == 0)) | ((k // 2 == 1) & (qi == 1)) \
        | ((k % 2 == 0) & (qj == 0)) | ((k % 2 == 1) & (qj == 1))
    z = a_ref[...] * s4 + t4
    z = jnp.where(z > 0, z, LEAKY_SLOPE * z)
    z = jnp.where(border, 0.0, z)
    acc_ref[...] += jnp.dot(z, b_ref[...], preferred_element_type=jnp.float32)

    @pl.when(k == pl.num_programs(1) - 1)
    def _():
        y = acc_ref[...] + bias_ref[...]
        o_ref[...] = 1.0 / (1.0 + jnp.exp(-y))


def kernel(x, w1, b1, w2, g2, be2, w3, g3, be3, w4, g4, be4, w5, b5):
    N = x.shape[0]

    xpad = jnp.pad(x.astype(jnp.float32), ((0, 0), (0, 0), (1, 1), (1, 1)))
    xq = xpad.reshape(N, 3, 17, 2, 34).transpose(0, 3, 1, 2, 4)
    w1f = w1.astype(jnp.float32)
    tsel = jnp.zeros((4, 3, 34, 16, 64), jnp.float32)
    ow = jnp.arange(16)
    for kw in range(4):
        upd = jnp.broadcast_to(
            jnp.transpose(w1f[:, :, :, kw], (2, 1, 0))[:, :, None, :],
            (4, 3, 16, 64))
        tsel = tsel.at[:, :, 2 * ow + kw, ow, :].add(upd)
    tsel = tsel.reshape(4, 102, 1024)
    bias1 = jnp.tile(b1.astype(jnp.float32), 16).reshape(1, 1024)

    nb1 = min(64, N)
    z2 = pl.pallas_call(
        functools.partial(_l1_kernel, nb=nb1),
        out_shape=jax.ShapeDtypeStruct((N, 9, 9, 256), jnp.float32),
        grid=(N // nb1,),
        in_specs=[
            pl.BlockSpec((nb1, 2, 3, 17, 34), lambda m: (m, 0, 0, 0, 0)),
            pl.BlockSpec((4, 102, 1024), lambda m: (0, 0, 0)),
            pl.BlockSpec((1, 1024), lambda m: (0, 0)),
        ],
        out_specs=pl.BlockSpec((nb1, 9, 9, 256), lambda m: (m, 0, 0, 0)),
        compiler_params=pltpu.CompilerParams(
            dimension_semantics=("parallel",)),
    )(xq, tsel, bias1)

    nb2 = min(32, N)
    z3, st2 = pl.pallas_call(
        functools.partial(_conv_kernel, Ho=8, nb=nb2),
        out_shape=(jax.ShapeDtypeStruct((N, 5, 5, 512), jnp.float32),
                   jax.ShapeDtypeStruct((N // nb2 * 8, 128), jnp.float32)),
        grid=(N // nb2,),
        in_specs=[
            pl.BlockSpec((nb2, 9, 9, 256), lambda m: (m, 0, 0, 0)),
            pl.BlockSpec((4, 256, 128), lambda m: (0, 0, 0)),
        ],
        out_specs=(pl.BlockSpec((nb2, 5, 5, 512), lambda m: (m, 0, 0, 0)),
                   pl.BlockSpec((8, 128), lambda m: (m, 0))),
        compiler_params=pltpu.CompilerParams(
            dimension_semantics=("parallel",)),
    )(z2, _wmat(w2, 64))
    nblk2 = N // nb2

    nb3 = min(64, N)
    z4, st3 = pl.pallas_call(
        functools.partial(_affine_conv_kernel, Ho=4, nb=nb3, C=128,
                          m_rows=float(N * 64)),
        out_shape=(jax.ShapeDtypeStruct((N, 3, 3, 1024), jnp.float32),
                   jax.ShapeDtypeStruct((N // nb3 * 8, 256), jnp.float32)),
        grid=(N // nb3,),
        in_specs=[
            pl.BlockSpec((nb3, 5, 5, 512), lambda m: (m, 0, 0, 0)),
            pl.BlockSpec((4, 512, 256), lambda m: (0, 0, 0)),
            pl.BlockSpec((nblk2 * 8, 128), lambda m: (0, 0)),
            pl.BlockSpec((1, 128), lambda m: (0, 0)),
            pl.BlockSpec((1, 128), lambda m: (0, 0)),
        ],
        out_specs=(pl.BlockSpec((nb3, 3, 3, 1024), lambda m: (m, 0, 0, 0)),
                   pl.BlockSpec((8, 256), lambda m: (m, 0))),
        compiler_params=pltpu.CompilerParams(
            dimension_semantics=("parallel",)),
    )(z3, _wmat(w3, 128), st2,
      g2.astype(jnp.float32).reshape(1, 128),
      be2.astype(jnp.float32).reshape(1, 128))
    nblk3 = N // nb3

    nb4 = min(64, N)
    z5, st4 = pl.pallas_call(
        functools.partial(_affine_conv_kernel, Ho=2, nb=nb4, C=256,
                          m_rows=float(N * 16)),
        out_shape=(jax.ShapeDtypeStruct((N, 8192), jnp.float32),
                   jax.ShapeDtypeStruct((N // nb4 * 8, 512), jnp.float32)),
        grid=(N // nb4,),
        in_specs=[
            pl.BlockSpec((nb4, 3, 3, 1024), lambda m: (m, 0, 0, 0)),
            pl.BlockSpec((4, 1024, 512), lambda m: (0, 0, 0)),
            pl.BlockSpec((nblk3 * 8, 256), lambda m: (0, 0)),
            pl.BlockSpec((1, 256), lambda m: (0, 0)),
            pl.BlockSpec((1, 256), lambda m: (0, 0)),
        ],
        out_specs=(pl.BlockSpec((nb4, 8192), lambda m: (m, 0)),
                   pl.BlockSpec((8, 512), lambda m: (m, 0))),
        compiler_params=pltpu.CompilerParams(
            dimension_semantics=("parallel",)),
    )(z4, _wmat(w4, 256), st3,
      g3.astype(jnp.float32).reshape(1, 256),
      be3.astype(jnp.float32).reshape(1, 256))
    nblk4 = N // nb4

    b5m = jnp.pad(_wmat(w5, 512).reshape(8192, 1), ((0, 0), (0, 127)))
    bias5 = jnp.pad(b5.astype(jnp.float32), (0, 127)).reshape(1, 128)

    nb5 = N // 2
    y = pl.pallas_call(
        functools.partial(_l5_kernel, m_rows=float(N * 4)),
        out_shape=jax.ShapeDtypeStruct((N, 128), jnp.float32),
        grid=(2, 4),
        in_specs=[
            pl.BlockSpec((nb5, 2048), lambda m, k: (m, k)),
            pl.BlockSpec((2048, 128), lambda m, k: (k, 0)),
            pl.BlockSpec((nblk4 * 8, 512), lambda m, k: (0, 0)),
            pl.BlockSpec((1, 512), lambda m, k: (0, 0)),
            pl.BlockSpec((1, 512), lambda m, k: (0, 0)),
            pl.BlockSpec((1, 128), lambda m, k: (0, 0)),
        ],
        out_specs=pl.BlockSpec((nb5, 128), lambda m, k: (m, 0)),
        scratch_shapes=[pltpu.VMEM((nb5, 128), jnp.float32)],
        compiler_params=pltpu.CompilerParams(
            dimension_semantics=("parallel", "arbitrary")),
    )(z5, b5m, st4,
      g4.astype(jnp.float32).reshape(1, 512),
      be4.astype(jnp.float32).reshape(1, 512), bias5)

    return y[:, :1].reshape(N, 1, 1, 1)
